# baseline (speedup 1.0000x reference)
import functools

import jax
import jax.numpy as jnp
from jax import lax
from jax.experimental import pallas as pl
from jax.experimental.pallas import tpu as pltpu
from jax.experimental.pallas import tpu_sc as plsc

_NE = 16
_H = 1024
_I = 2048
_T = 128
_NJ = 2
_BI = _I // _NJ

_SC_CORES = 1
_SC_WORKERS = 16 * _SC_CORES
_TPW = _T // _SC_WORKERS


def _logits_body(x_ref, rw_ref, out_ref):
    out_ref[...] = jax.lax.dot_general(
        x_ref[...], rw_ref[...], (((1,), (1,)), ((), ())),
        preferred_element_type=jnp.float32)


def _router_logits(x, rw):
    return pl.pallas_call(
        _logits_body,
        out_shape=jax.ShapeDtypeStruct((_T, _NE), jnp.float32),
    )(x, rw)


def _sc_router_body(lg_hbm, dw_hbm, lv, ov, sem):
    wid = lax.axis_index("s") * _SC_CORES + lax.axis_index("c")
    base = wid * _TPW
    pltpu.async_copy(lg_hbm.at[pl.ds(base, _TPW)], lv, sem).wait()
    idx = lax.iota(jnp.int32, _NE)
    for i in range(_TPW):
        v = lv[i]
        m = jnp.max(v)
        ex = jnp.exp(v - m)
        probs = ex / jnp.sum(ex)
        pmax = jnp.max(probs)
        first = jnp.min(jnp.where(probs >= pmax, idx, _NE))
        ov[i] = jnp.where(idx == first, pmax, 0.0)
    pltpu.async_copy(ov, dw_hbm.at[pl.ds(base, _TPW)], sem).wait()


def _sc_router(logits):
    mesh = plsc.VectorSubcoreMesh(core_axis_name="c", subcore_axis_name="s",
                                  num_cores=_SC_CORES)
    kern = functools.partial(
        pl.kernel,
        out_type=jax.ShapeDtypeStruct((_T, _NE), jnp.float32),
        mesh=mesh,
        scratch_types=[
            pltpu.VMEM((_TPW, _NE), jnp.float32),
            pltpu.VMEM((_TPW, _NE), jnp.float32),
            pltpu.SemaphoreType.DMA,
        ],
        compiler_params=pltpu.CompilerParams(needs_layout_passes=False),
    )(_sc_router_body)
    return kern(logits)


def _moe_body(x_ref, dw_ref, wg_ref, wu_ref, w2s_ref, out_ref):
    e = pl.program_id(0)
    j = pl.program_id(1)

    @pl.when((e == 0) & (j == 0))
    def _init():
        out_ref[...] = jnp.zeros_like(out_ref)

    xb = x_ref[...].astype(jnp.bfloat16)
    gate = jax.lax.dot_general(
        xb, wg_ref[0].astype(jnp.bfloat16), (((1,), (1,)), ((), ())),
        preferred_element_type=jnp.float32)
    up = jax.lax.dot_general(
        xb, wu_ref[0].astype(jnp.bfloat16), (((1,), (1,)), ((), ())),
        preferred_element_type=jnp.float32)
    act = (gate * jax.lax.logistic(gate)) * up
    eids = jax.lax.broadcasted_iota(jnp.int32, (_T, _NE), 1)
    we = jnp.sum(jnp.where(eids == e, dw_ref[...], 0.0), axis=1,
                 keepdims=True)
    actb = (act * we).astype(jnp.bfloat16)
    contrib = jax.lax.dot_general(
        actb, w2s_ref[0].astype(jnp.bfloat16), (((1,), (1,)), ((), ())),
        preferred_element_type=jnp.float32)
    out_ref[...] += contrib


def _run_experts(x, dw, ws, w2s):
    return pl.pallas_call(
        _moe_body,
        grid=(_NE, _NJ),
        in_specs=[
            pl.BlockSpec((_T, _H), lambda e, j: (0, 0)),
            pl.BlockSpec((_T, _NE), lambda e, j: (0, 0)),
            pl.BlockSpec((1, _BI, _H), lambda e, j: (e, j, 0)),
            pl.BlockSpec((1, _BI, _H), lambda e, j: (e, j + _NJ, 0)),
            pl.BlockSpec((1, _H, _BI), lambda e, j: (e, 0, j)),
        ],
        out_specs=pl.BlockSpec((_T, _H), lambda e, j: (0, 0)),
        out_shape=jax.ShapeDtypeStruct((_T, _H), jnp.float32),
        compiler_params=pltpu.CompilerParams(
            dimension_semantics=("arbitrary", "arbitrary")),
    )(x, dw, ws, ws, w2s)


def kernel(hidden_states, router_w, ws, w2s, top_k):
    logits = hidden_states @ router_w.T
    dw = _sc_router(logits)
    out = _run_experts(hidden_states, dw, ws, w2s)
    return out * (jnp.asarray(top_k, jnp.float32) / 1.0)


# speedup vs baseline: 1.0121x; 1.0121x over previous
import functools

import jax
import jax.numpy as jnp
from jax import lax
from jax.experimental import pallas as pl
from jax.experimental.pallas import tpu as pltpu
from jax.experimental.pallas import tpu_sc as plsc

_NE = 16
_H = 1024
_I = 2048
_T = 128
_NJ = 2
_BI = _I // _NJ

_SC_CORES = 1
_SC_WORKERS = 16 * _SC_CORES
_TPW = _T // _SC_WORKERS


def _logits_body(x_ref, rw_ref, out_ref):
    out_ref[...] = jax.lax.dot_general(
        x_ref[...], rw_ref[...], (((1,), (1,)), ((), ())),
        preferred_element_type=jnp.float32)


def _router_logits(x, rw):
    return pl.pallas_call(
        _logits_body,
        out_shape=jax.ShapeDtypeStruct((_T, _NE), jnp.float32),
    )(x, rw)


def _sc_router_body(lg_hbm, dw_hbm, lv, ov, sem):
    wid = lax.axis_index("s") * _SC_CORES + lax.axis_index("c")
    base = wid * _TPW
    pltpu.async_copy(lg_hbm.at[pl.ds(base, _TPW)], lv, sem).wait()
    idx = lax.iota(jnp.int32, _NE)
    for i in range(_TPW):
        v = lv[i]
        m = jnp.max(v)
        ex = jnp.exp(v - m)
        probs = ex / jnp.sum(ex)
        pmax = jnp.max(probs)
        first = jnp.min(jnp.where(probs >= pmax, idx, _NE))
        ov[i] = jnp.where(idx == first, pmax, 0.0)
    pltpu.async_copy(ov, dw_hbm.at[pl.ds(base, _TPW)], sem).wait()


def _sc_router(logits):
    mesh = plsc.VectorSubcoreMesh(core_axis_name="c", subcore_axis_name="s",
                                  num_cores=_SC_CORES)
    kern = functools.partial(
        pl.kernel,
        out_type=jax.ShapeDtypeStruct((_T, _NE), jnp.float32),
        mesh=mesh,
        scratch_types=[
            pltpu.VMEM((_TPW, _NE), jnp.float32),
            pltpu.VMEM((_TPW, _NE), jnp.float32),
            pltpu.SemaphoreType.DMA,
        ],
        compiler_params=pltpu.CompilerParams(needs_layout_passes=False),
    )(_sc_router_body)
    return kern(logits)


def _moe_body(x_ref, dw_ref, wg_ref, wu_ref, w2s_ref, out_ref):
    e = pl.program_id(0)
    j = pl.program_id(1)

    @pl.when((e == 0) & (j == 0))
    def _init():
        out_ref[...] = jnp.zeros_like(out_ref)

    xb = x_ref[...].astype(jnp.bfloat16)
    gate = jax.lax.dot_general(
        xb, wg_ref[0].astype(jnp.bfloat16), (((1,), (1,)), ((), ())),
        preferred_element_type=jnp.float32)
    up = jax.lax.dot_general(
        xb, wu_ref[0].astype(jnp.bfloat16), (((1,), (1,)), ((), ())),
        preferred_element_type=jnp.float32)
    act = (gate * jax.lax.logistic(gate)) * up
    eids = jax.lax.broadcasted_iota(jnp.int32, (_T, _NE), 1)
    we = jnp.sum(jnp.where(eids == e, dw_ref[...], 0.0), axis=1,
                 keepdims=True)
    actb = (act * we).astype(jnp.bfloat16)
    contrib = jax.lax.dot_general(
        actb, w2s_ref[0].astype(jnp.bfloat16), (((1,), (1,)), ((), ())),
        preferred_element_type=jnp.float32)
    out_ref[...] += contrib


def _run_experts(x, dw, ws, w2s):
    return pl.pallas_call(
        _moe_body,
        grid=(_NE, _NJ),
        in_specs=[
            pl.BlockSpec((_T, _H), lambda e, j: (0, 0)),
            pl.BlockSpec((_T, _NE), lambda e, j: (0, 0)),
            pl.BlockSpec((1, _BI, _H), lambda e, j: (e, j, 0)),
            pl.BlockSpec((1, _BI, _H), lambda e, j: (e, j + _NJ, 0)),
            pl.BlockSpec((1, _H, _BI), lambda e, j: (e, 0, j)),
        ],
        out_specs=pl.BlockSpec((_T, _H), lambda e, j: (0, 0)),
        out_shape=jax.ShapeDtypeStruct((_T, _H), jnp.float32),
        compiler_params=pltpu.CompilerParams(
            dimension_semantics=("arbitrary", "arbitrary")),
    )(x, dw, ws, ws, w2s)


def kernel(hidden_states, router_w, ws, w2s, top_k):
    logits = _router_logits(hidden_states, router_w)
    dw = _sc_router(logits)
    out = _run_experts(hidden_states, dw, ws, w2s)
    return out * (jnp.asarray(top_k, jnp.float32) / 1.0)
